# baseline (speedup 1.0000x reference)
import functools

import jax
import jax.numpy as jnp
from jax import lax
from jax.experimental import pallas as pl
from jax.experimental.pallas import tpu as pltpu
from jax.experimental.pallas import tpu_sc as plsc

_NBUF = 8
_L = 16
_IPW = 16


def _sc_route(cat_hbm, gam_hbm, bet_hbm, dep_hbm, ga_hbm, gd_hbm,
              s_hbm, t_hbm,
              cat_v, gam_v, bet_v, dep_v, ga_v, gd_v, s_v, t_v,
              *, k, nw_active):
    wid = lax.axis_index("s") * 2 + lax.axis_index("c")

    @pl.when(wid < nw_active)
    def _():
        pltpu.sync_copy(cat_hbm, cat_v)
        pltpu.sync_copy(gam_hbm, gam_v)
        pltpu.sync_copy(bet_hbm, bet_v)
        pltpu.sync_copy(dep_hbm, dep_v)
        pltpu.sync_copy(ga_hbm.at[wid], ga_v)
        pltpu.sync_copy(gd_hbm.at[wid], gd_v)

        iota = lax.iota(jnp.int32, _L)
        s_acc = jnp.zeros((_L,), jnp.float32)
        t_acc = jnp.zeros((_L,), jnp.float32)
        for j in range(_IPW):
            A = [jnp.float32(1.0)]
            C = [jnp.float32(0.0)]
            for i in range(k):
                zv = cat_v[i] + ga_v[j, i]
                mx = lax.reduce_max(zv, (0,))
                am = lax.reduce_min(jnp.where(zv >= mx, iota, _L), (0,))
                oh = iota == am
                si = lax.reduce_sum(jnp.where(oh, gam_v[i], 0.0), (0,))
                ti = lax.reduce_sum(jnp.where(oh, bet_v[i], 0.0), (0,))
                A.append(si * A[-1])
                C.append(si * C[-1] + ti)
            zd = dep_v[...] + gd_v[j]
            mxd = lax.reduce_max(zd, (0,))
            m = lax.reduce_min(jnp.where(zd >= mxd, iota, _L), (0,))
            S = A[k]
            T = C[k]
            for i in range(k - 1, -1, -1):
                sel = m == i
                S = jnp.where(sel, A[i], S)
                T = jnp.where(sel, C[i], T)
            lane = iota == j
            s_acc = jnp.where(lane, S, s_acc)
            t_acc = jnp.where(lane, T, t_acc)
        s_v[...] = s_acc
        t_v[...] = t_acc
        pltpu.sync_copy(s_v, s_hbm.at[wid])
        pltpu.sync_copy(t_v, t_hbm.at[wid])


def _tc_body(s_ref, t_ref, x_hbm, o_hbm, ibuf, obuf, in_sems, out_sems, *, B):
    for j in range(_NBUF):
        pltpu.make_async_copy(x_hbm.at[j], ibuf.at[j], in_sems.at[j]).start()
    for i in range(B):
        slot = i % _NBUF
        pltpu.make_async_copy(x_hbm.at[i], ibuf.at[slot], in_sems.at[slot]).wait()
        if i >= _NBUF:
            pltpu.make_async_copy(
                obuf.at[slot], o_hbm.at[i - _NBUF], out_sems.at[slot]).wait()
        s = s_ref[0:1, i:i + 1].reshape(1, 1, 1)
        t = t_ref[0:1, i:i + 1].reshape(1, 1, 1)
        obuf[slot] = s * ibuf[slot] + t
        pltpu.make_async_copy(obuf.at[slot], o_hbm.at[i], out_sems.at[slot]).start()
        nxt = i + _NBUF
        if nxt < B:
            pltpu.make_async_copy(
                x_hbm.at[nxt], ibuf.at[slot], in_sems.at[slot]).start()
    for i in range(B - _NBUF, B):
        slot = i % _NBUF
        pltpu.make_async_copy(obuf.at[slot], o_hbm.at[i], out_sems.at[slot]).wait()


def kernel(input, cat_logits, gammas, betas_aug, depth_logits):
    B = input.shape[0]
    k, d = cat_logits.shape
    C = input.shape[1]
    H, W = 392, 128
    x4 = input.reshape(B, C, H, W)

    key = jax.random.key(42)
    k_aug, k_depth = jax.random.split(key)
    ua = jax.random.uniform(k_aug, (B, k, d), minval=1e-6, maxval=1.0 - 1e-6)
    ud = jax.random.uniform(k_depth, (B, k + 1), minval=1e-6, maxval=1.0 - 1e-6)
    ga = -jnp.log(-jnp.log(ua))
    gd = -jnp.log(-jnp.log(ud))

    neg = jnp.float32(-1e30)
    gap = jnp.pad(ga, ((0, 0), (0, 0), (0, _L - d)), constant_values=neg)
    gdp = jnp.pad(gd, ((0, 0), (0, _L - (k + 1))), constant_values=neg)
    catp = jnp.pad(cat_logits, ((0, 0), (0, _L - d)), constant_values=neg)
    gamp = jnp.pad(gammas, ((0, 0), (0, _L - d)))
    betp = jnp.pad(betas_aug, ((0, 0), (0, _L - d)))
    depp = jnp.pad(depth_logits, (0, _L - (k + 1)), constant_values=neg)

    nw_active = B // _IPW
    mesh = plsc.VectorSubcoreMesh(core_axis_name="c", subcore_axis_name="s")
    route = functools.partial(
        pl.kernel,
        mesh=mesh,
        compiler_params=pltpu.CompilerParams(needs_layout_passes=False, use_tc_tiling_on_sc=False),
        out_type=(jax.ShapeDtypeStruct((B // _IPW, _IPW), jnp.float32),
                  jax.ShapeDtypeStruct((B // _IPW, _IPW), jnp.float32)),
        scratch_types=[
            pltpu.VMEM((k, _L), jnp.float32),
            pltpu.VMEM((k, _L), jnp.float32),
            pltpu.VMEM((k, _L), jnp.float32),
            pltpu.VMEM((_L,), jnp.float32),
            pltpu.VMEM((_IPW, k, _L), jnp.float32),
            pltpu.VMEM((_IPW, _L), jnp.float32),
            pltpu.VMEM((_L,), jnp.float32),
            pltpu.VMEM((_L,), jnp.float32),
        ],
    )(functools.partial(_sc_route, k=k, nw_active=nw_active))
    gap = gap.reshape(nw_active, _IPW, k, _L)
    gdp = gdp.reshape(nw_active, _IPW, _L)
    s_tab, t_tab = route(catp, gamp, betp, depp, gap, gdp)

    out = pl.pallas_call(
        functools.partial(_tc_body, B=B),
        in_specs=[
            pl.BlockSpec(memory_space=pltpu.MemorySpace.VMEM),
            pl.BlockSpec(memory_space=pltpu.MemorySpace.VMEM),
            pl.BlockSpec(memory_space=pltpu.MemorySpace.HBM),
        ],
        out_specs=pl.BlockSpec(memory_space=pltpu.MemorySpace.HBM),
        out_shape=jax.ShapeDtypeStruct((B, C, H, W), jnp.float32),
        scratch_shapes=[
            pltpu.VMEM((_NBUF, C, H, W), jnp.float32),
            pltpu.VMEM((_NBUF, C, H, W), jnp.float32),
            pltpu.SemaphoreType.DMA((_NBUF,)),
            pltpu.SemaphoreType.DMA((_NBUF,)),
        ],
    )(s_tab.reshape(1, B), t_tab.reshape(1, B), x4)
    return out.reshape(input.shape)
